# baseline (speedup 1.0000x reference)
import functools

import jax
import jax.numpy as jnp
from jax import lax
from jax.experimental import pallas as pl
from jax.experimental.pallas import tpu as pltpu
from jax.experimental.pallas import tpu_sc as plsc

N = 10000
N_ACC = 10112
E = 320000
D = 128
CHUNK = 80
ROWS = E // CHUNK
NW = 32
KPW = ROWS // NW


def _sc_aggregate(h, src1d, dst1d, attr2d, zeros):
    mesh = plsc.VectorSubcoreMesh(core_axis_name="c", subcore_axis_name="s")

    @functools.partial(
        pl.kernel,
        mesh=mesh,
        out_type=jax.ShapeDtypeStruct((2, N_ACC, D), jnp.float32),
        scratch_types=[
            pltpu.VMEM((2, CHUNK), jnp.int32),
            pltpu.VMEM((2, CHUNK), jnp.int32),
            pltpu.VMEM((2, CHUNK, D), jnp.float32),
            pltpu.VMEM((2, CHUNK, D), jnp.float32),
            pltpu.VMEM_SHARED((N_ACC, D), jnp.float32),
            pltpu.SemaphoreType.DMA,
            pltpu.SemaphoreType.DMA,
            pltpu.SemaphoreType.DMA,
            pltpu.SemaphoreType.DMA,
        ],
    )
    def k(h_hbm, src_hbm, dst_hbm, attr_hbm, z_hbm, out_hbm,
          src_v, dst_v, hrows_v, attr_v, aggr_sh, sem_g, sem_a, sem_i,
          sem_s):
        c = lax.axis_index("c")
        s = lax.axis_index("s")
        w = c * 16 + s

        rows_per_tile = N_ACC // 16
        pltpu.sync_copy(z_hbm.at[pl.ds(s * rows_per_tile, rows_per_tile)],
                        aggr_sh.at[pl.ds(s * rows_per_tile, rows_per_tile)])
        plsc.subcore_barrier()

        start = w * KPW
        end = start + KPW

        def issue_idx(r):
            p = r % 2
            sl = pl.ds(r * CHUNK, CHUNK)
            pltpu.async_copy(src_hbm.at[sl], src_v.at[p], sem_i)
            pltpu.async_copy(dst_hbm.at[sl], dst_v.at[p], sem_i)

        def wait_idx(r):
            p = r % 2
            sl = pl.ds(r * CHUNK, CHUNK)
            pltpu.make_async_copy(src_hbm.at[sl], src_v.at[p], sem_i).wait()
            pltpu.make_async_copy(dst_hbm.at[sl], dst_v.at[p], sem_i).wait()

        def issue_gather(r):
            p = r % 2
            pltpu.async_copy(h_hbm.at[src_v.at[p]], hrows_v.at[p], sem_g)

        def wait_gather(r):
            p = r % 2
            pltpu.make_async_copy(h_hbm.at[src_v.at[p]], hrows_v.at[p],
                                  sem_g).wait()

        def issue_attr(r):
            p = r % 2
            sl = pl.ds(r * CHUNK, CHUNK)
            pltpu.async_copy(attr_hbm.at[sl], attr_v.at[p], sem_a)

        def wait_attr(r):
            p = r % 2
            sl = pl.ds(r * CHUNK, CHUNK)
            pltpu.make_async_copy(attr_hbm.at[sl], attr_v.at[p],
                                  sem_a).wait()

        p0 = start % 2
        sl0 = pl.ds(start * CHUNK, CHUNK)
        pltpu.sync_copy(src_hbm.at[sl0], src_v.at[p0])
        pltpu.sync_copy(dst_hbm.at[sl0], dst_v.at[p0])
        issue_attr(start)
        issue_gather(start)

        def wait_scatter(r):
            p = r % 2
            pltpu.make_async_copy(attr_v.at[p], aggr_sh.at[dst_v.at[p]],
                                  sem_s).wait()

        def body(r, carry):
            p = r % 2

            @pl.when(r > start)
            def _():
                wait_scatter(r - 1)

            @pl.when(r + 1 < end)
            def _():
                issue_idx(r + 1)
                issue_attr(r + 1)

            wait_attr(r)
            wait_gather(r)

            @pl.when(r + 1 < end)
            def _():
                wait_idx(r + 1)
                issue_gather(r + 1)

            def row_body(i, carry2):
                for j in range(D // 16):
                    sl = pl.ds(j * 16, 16)
                    attr_v[p, i, sl] = attr_v[p, i, sl] + hrows_v[p, i, sl]
                return carry2

            lax.fori_loop(0, CHUNK, row_body, 0)

            pltpu.async_copy(attr_v.at[p], aggr_sh.at[dst_v.at[p]], sem_s,
                             add=True)
            return carry


        lax.fori_loop(start, end, body, 0)
        wait_scatter(end - 1)
        plsc.subcore_barrier()

        pltpu.sync_copy(aggr_sh.at[pl.ds(s * rows_per_tile, rows_per_tile)],
                        out_hbm.at[c, pl.ds(s * rows_per_tile, rows_per_tile)])

    return k(h, src1d, dst1d, attr2d, zeros)


def _tc_mlp_body(h_ref, p_ref, w1_ref, b1_ref, w2_ref, b2_ref, eps_ref,
                 out_ref):
    aggr = p_ref[0] + p_ref[1]
    hid = jnp.dot(aggr, w1_ref[...], preferred_element_type=jnp.float32)
    hid = jnp.maximum(hid + b1_ref[...], 0.0)
    out = jnp.dot(hid, w2_ref[...], preferred_element_type=jnp.float32)
    out_ref[...] = (1.0 + eps_ref[0]) * h_ref[...] + out + b2_ref[...]


def _tc_mlp(h, partials, W1, b1, W2, b2, eps):
    BR = 256
    grid = (pl.cdiv(N, BR),)
    return pl.pallas_call(
        _tc_mlp_body,
        grid=grid,
        in_specs=[
            pl.BlockSpec((BR, D), lambda i: (i, 0)),
            pl.BlockSpec((2, BR, D), lambda i: (0, i, 0)),
            pl.BlockSpec((D, 2 * D), lambda i: (0, 0)),
            pl.BlockSpec((1, 2 * D), lambda i: (0, 0)),
            pl.BlockSpec((2 * D, D), lambda i: (0, 0)),
            pl.BlockSpec((1, D), lambda i: (0, 0)),
            pl.BlockSpec(memory_space=pltpu.SMEM),
        ],
        out_specs=pl.BlockSpec((BR, D), lambda i: (i, 0)),
        out_shape=jax.ShapeDtypeStruct((N, D), jnp.float32),
    )(h, partials, W1, b1.reshape(1, 2 * D), W2, b2.reshape(1, D), eps)


def kernel(h, edge_index, edge_attr, W1, b1, W2, b2, eps):
    src1d = edge_index[0]
    dst1d = edge_index[1]
    zeros = jnp.zeros((N_ACC, D), jnp.float32)
    partials = _sc_aggregate(h, src1d, dst1d, edge_attr, zeros)
    return _tc_mlp(h, partials, W1, b1, W2, b2, eps)


# speedup vs baseline: 1.0806x; 1.0806x over previous
import functools

import jax
import jax.numpy as jnp
from jax import lax
from jax.experimental import pallas as pl
from jax.experimental.pallas import tpu as pltpu
from jax.experimental.pallas import tpu_sc as plsc

N = 10000
N_ACC = 10112
E = 320000
D = 128
CHUNK = 80
ROWS = E // CHUNK
NW = 32
KPW = ROWS // NW


def _sc_aggregate(h, src1d, dst1d, attr2d, zeros):
    mesh = plsc.VectorSubcoreMesh(core_axis_name="c", subcore_axis_name="s")

    @functools.partial(
        pl.kernel,
        mesh=mesh,
        out_type=jax.ShapeDtypeStruct((2, N_ACC, D), jnp.float32),
        scratch_types=[
            pltpu.VMEM((2, CHUNK), jnp.int32),
            pltpu.VMEM((2, CHUNK), jnp.int32),
            pltpu.VMEM((2, CHUNK, D), jnp.float32),
            pltpu.VMEM((2, CHUNK, D), jnp.float32),
            pltpu.VMEM_SHARED((N_ACC, D), jnp.float32),
            pltpu.SemaphoreType.DMA,
            pltpu.SemaphoreType.DMA,
            pltpu.SemaphoreType.DMA,
            pltpu.SemaphoreType.DMA,
        ],
    )
    def k(h_hbm, src_hbm, dst_hbm, attr_hbm, z_hbm, out_hbm,
          src_v, dst_v, hrows_v, attr_v, aggr_sh, sem_g, sem_a, sem_i,
          sem_s):
        c = lax.axis_index("c")
        s = lax.axis_index("s")
        w = c * 16 + s

        rows_per_tile = N_ACC // 16
        pltpu.sync_copy(z_hbm.at[pl.ds(s * rows_per_tile, rows_per_tile)],
                        aggr_sh.at[pl.ds(s * rows_per_tile, rows_per_tile)])
        plsc.subcore_barrier()

        start = w * KPW
        end = start + KPW

        def issue_idx(r):
            p = r % 2
            sl = pl.ds(r * CHUNK, CHUNK)
            pltpu.async_copy(src_hbm.at[sl], src_v.at[p], sem_i)
            pltpu.async_copy(dst_hbm.at[sl], dst_v.at[p], sem_i)

        def wait_idx(r):
            p = r % 2
            sl = pl.ds(r * CHUNK, CHUNK)
            pltpu.make_async_copy(src_hbm.at[sl], src_v.at[p], sem_i).wait()
            pltpu.make_async_copy(dst_hbm.at[sl], dst_v.at[p], sem_i).wait()

        def issue_gather(r):
            p = r % 2
            pltpu.async_copy(h_hbm.at[src_v.at[p]], hrows_v.at[p], sem_g)

        def wait_gather(r):
            p = r % 2
            pltpu.make_async_copy(h_hbm.at[src_v.at[p]], hrows_v.at[p],
                                  sem_g).wait()

        def issue_attr(r):
            p = r % 2
            sl = pl.ds(r * CHUNK, CHUNK)
            pltpu.async_copy(attr_hbm.at[sl], attr_v.at[p], sem_a)

        def wait_attr(r):
            p = r % 2
            sl = pl.ds(r * CHUNK, CHUNK)
            pltpu.make_async_copy(attr_hbm.at[sl], attr_v.at[p],
                                  sem_a).wait()

        p0 = start % 2
        sl0 = pl.ds(start * CHUNK, CHUNK)
        pltpu.sync_copy(src_hbm.at[sl0], src_v.at[p0])
        pltpu.sync_copy(dst_hbm.at[sl0], dst_v.at[p0])
        issue_attr(start)
        issue_gather(start)

        def wait_scatter(r):
            p = r % 2
            pltpu.make_async_copy(attr_v.at[p], aggr_sh.at[dst_v.at[p]],
                                  sem_s).wait()

        def body(r, carry):
            p = r % 2

            @pl.when(r > start)
            def _():
                wait_scatter(r - 1)

            @pl.when(r + 1 < end)
            def _():
                issue_idx(r + 1)
                issue_attr(r + 1)

            wait_attr(r)
            wait_gather(r)

            @pl.when(r + 1 < end)
            def _():
                wait_idx(r + 1)
                issue_gather(r + 1)

            def row_body(i, carry2):
                base = i * 8
                for jj in range(8):
                    for j in range(D // 16):
                        sl = pl.ds(j * 16, 16)
                        attr_v[p, base + jj, sl] = (
                            attr_v[p, base + jj, sl]
                            + hrows_v[p, base + jj, sl])
                return carry2

            lax.fori_loop(0, CHUNK // 8, row_body, 0)

            pltpu.async_copy(attr_v.at[p], aggr_sh.at[dst_v.at[p]], sem_s,
                             add=True)
            return carry


        lax.fori_loop(start, end, body, 0)
        wait_scatter(end - 1)
        plsc.subcore_barrier()

        pltpu.sync_copy(aggr_sh.at[pl.ds(s * rows_per_tile, rows_per_tile)],
                        out_hbm.at[c, pl.ds(s * rows_per_tile, rows_per_tile)])

    return k(h, src1d, dst1d, attr2d, zeros)


def _tc_mlp_body(h_ref, p_ref, w1_ref, b1_ref, w2_ref, b2_ref, eps_ref,
                 out_ref):
    aggr = p_ref[0] + p_ref[1]
    hid = jnp.dot(aggr, w1_ref[...], preferred_element_type=jnp.float32)
    hid = jnp.maximum(hid + b1_ref[...], 0.0)
    out = jnp.dot(hid, w2_ref[...], preferred_element_type=jnp.float32)
    out_ref[...] = (1.0 + eps_ref[0]) * h_ref[...] + out + b2_ref[...]


def _tc_mlp(h, partials, W1, b1, W2, b2, eps):
    BR = 256
    grid = (pl.cdiv(N, BR),)
    return pl.pallas_call(
        _tc_mlp_body,
        grid=grid,
        in_specs=[
            pl.BlockSpec((BR, D), lambda i: (i, 0)),
            pl.BlockSpec((2, BR, D), lambda i: (0, i, 0)),
            pl.BlockSpec((D, 2 * D), lambda i: (0, 0)),
            pl.BlockSpec((1, 2 * D), lambda i: (0, 0)),
            pl.BlockSpec((2 * D, D), lambda i: (0, 0)),
            pl.BlockSpec((1, D), lambda i: (0, 0)),
            pl.BlockSpec(memory_space=pltpu.SMEM),
        ],
        out_specs=pl.BlockSpec((BR, D), lambda i: (i, 0)),
        out_shape=jax.ShapeDtypeStruct((N, D), jnp.float32),
    )(h, partials, W1, b1.reshape(1, 2 * D), W2, b2.reshape(1, D), eps)


def kernel(h, edge_index, edge_attr, W1, b1, W2, b2, eps):
    src1d = edge_index[0]
    dst1d = edge_index[1]
    zeros = jnp.zeros((N_ACC, D), jnp.float32)
    partials = _sc_aggregate(h, src1d, dst1d, edge_attr, zeros)
    return _tc_mlp(h, partials, W1, b1, W2, b2, eps)


# speedup vs baseline: 2.6826x; 2.4824x over previous
import functools

import jax
import jax.numpy as jnp
from jax import lax
from jax.experimental import pallas as pl
from jax.experimental.pallas import tpu as pltpu
from jax.experimental.pallas import tpu_sc as plsc

N = 10000
N_ACC = 10112
E = 320000
D = 128
CHUNK = 80
ROWS = E // CHUNK
NW = 32
KPW = ROWS // NW


def _sc_aggregate(h, src1d, dst1d, attr2d, zeros):
    mesh = plsc.VectorSubcoreMesh(core_axis_name="c", subcore_axis_name="s")

    @functools.partial(
        pl.kernel,
        mesh=mesh,
        out_type=jax.ShapeDtypeStruct((2, N_ACC, D), jnp.float32),
        scratch_types=[
            pltpu.VMEM((2, CHUNK), jnp.int32),
            pltpu.VMEM((2, CHUNK), jnp.int32),
            pltpu.VMEM((2, CHUNK, D), jnp.float32),
            pltpu.VMEM((2, CHUNK, D), jnp.float32),
            pltpu.VMEM_SHARED((N_ACC, D), jnp.float32),
            pltpu.SemaphoreType.DMA,
            pltpu.SemaphoreType.DMA,
            pltpu.SemaphoreType.DMA,
        ],
    )
    def k(h_hbm, src_hbm, dst_hbm, attr_hbm, z_hbm, out_hbm,
          src_v, dst_v, hrows_v, attr_v, aggr_sh, sem_g, sem_a, sem_i):
        c = lax.axis_index("c")
        s = lax.axis_index("s")
        w = c * 16 + s

        rows_per_tile = N_ACC // 16
        pltpu.sync_copy(z_hbm.at[pl.ds(s * rows_per_tile, rows_per_tile)],
                        aggr_sh.at[pl.ds(s * rows_per_tile, rows_per_tile)])
        plsc.subcore_barrier()

        start = w * KPW
        end = start + KPW

        def issue_idx(r):
            p = r % 2
            sl = pl.ds(r * CHUNK, CHUNK)
            pltpu.async_copy(src_hbm.at[sl], src_v.at[p], sem_i)
            pltpu.async_copy(dst_hbm.at[sl], dst_v.at[p], sem_i)

        def wait_idx(r):
            p = r % 2
            sl = pl.ds(r * CHUNK, CHUNK)
            pltpu.make_async_copy(src_hbm.at[sl], src_v.at[p], sem_i).wait()
            pltpu.make_async_copy(dst_hbm.at[sl], dst_v.at[p], sem_i).wait()

        def issue_gather(r):
            p = r % 2
            pltpu.async_copy(h_hbm.at[src_v.at[p]], hrows_v.at[p], sem_g)

        def wait_gather(r):
            p = r % 2
            pltpu.make_async_copy(h_hbm.at[src_v.at[p]], hrows_v.at[p],
                                  sem_g).wait()

        def issue_attr(r):
            p = r % 2
            sl = pl.ds(r * CHUNK, CHUNK)
            pltpu.async_copy(attr_hbm.at[sl], attr_v.at[p], sem_a)

        def wait_attr(r):
            p = r % 2
            sl = pl.ds(r * CHUNK, CHUNK)
            pltpu.make_async_copy(attr_hbm.at[sl], attr_v.at[p],
                                  sem_a).wait()

        p0 = start % 2
        sl0 = pl.ds(start * CHUNK, CHUNK)
        pltpu.sync_copy(src_hbm.at[sl0], src_v.at[p0])
        pltpu.sync_copy(dst_hbm.at[sl0], dst_v.at[p0])
        issue_attr(start)
        issue_gather(start)

        def body(r, carry):
            p = r % 2

            @pl.when(r + 1 < end)
            def _():
                issue_idx(r + 1)
                issue_attr(r + 1)

            wait_attr(r)
            pltpu.sync_copy(attr_v.at[p], aggr_sh.at[dst_v.at[p]], add=True)
            wait_gather(r)

            @pl.when(r + 1 < end)
            def _():
                wait_idx(r + 1)
                issue_gather(r + 1)

            pltpu.sync_copy(hrows_v.at[p], aggr_sh.at[dst_v.at[p]], add=True)
            return carry

        lax.fori_loop(start, end, body, 0)
        plsc.subcore_barrier()

        pltpu.sync_copy(aggr_sh.at[pl.ds(s * rows_per_tile, rows_per_tile)],
                        out_hbm.at[c, pl.ds(s * rows_per_tile, rows_per_tile)])

    return k(h, src1d, dst1d, attr2d, zeros)


def _tc_mlp_body(h_ref, p_ref, w1_ref, b1_ref, w2_ref, b2_ref, eps_ref,
                 out_ref):
    aggr = p_ref[0] + p_ref[1]
    hid = jnp.dot(aggr, w1_ref[...], preferred_element_type=jnp.float32)
    hid = jnp.maximum(hid + b1_ref[...], 0.0)
    out = jnp.dot(hid, w2_ref[...], preferred_element_type=jnp.float32)
    out_ref[...] = (1.0 + eps_ref[0]) * h_ref[...] + out + b2_ref[...]


def _tc_mlp(h, partials, W1, b1, W2, b2, eps):
    BR = 256
    grid = (pl.cdiv(N, BR),)
    return pl.pallas_call(
        _tc_mlp_body,
        grid=grid,
        in_specs=[
            pl.BlockSpec((BR, D), lambda i: (i, 0)),
            pl.BlockSpec((2, BR, D), lambda i: (0, i, 0)),
            pl.BlockSpec((D, 2 * D), lambda i: (0, 0)),
            pl.BlockSpec((1, 2 * D), lambda i: (0, 0)),
            pl.BlockSpec((2 * D, D), lambda i: (0, 0)),
            pl.BlockSpec((1, D), lambda i: (0, 0)),
            pl.BlockSpec(memory_space=pltpu.SMEM),
        ],
        out_specs=pl.BlockSpec((BR, D), lambda i: (i, 0)),
        out_shape=jax.ShapeDtypeStruct((N, D), jnp.float32),
    )(h, partials, W1, b1.reshape(1, 2 * D), W2, b2.reshape(1, D), eps)


def kernel(h, edge_index, edge_attr, W1, b1, W2, b2, eps):
    src1d = edge_index[0]
    dst1d = edge_index[1]
    zeros = jnp.zeros((N_ACC, D), jnp.float32)
    partials = _sc_aggregate(h, src1d, dst1d, edge_attr, zeros)
    return _tc_mlp(h, partials, W1, b1, W2, b2, eps)


# speedup vs baseline: 2.7388x; 1.0209x over previous
import functools

import jax
import jax.numpy as jnp
from jax import lax
from jax.experimental import pallas as pl
from jax.experimental.pallas import tpu as pltpu
from jax.experimental.pallas import tpu_sc as plsc

N = 10000
N_ACC = 10112
E = 320000
D = 128
CHUNK = 80
ROWS = E // CHUNK
NW = 32
KPW = ROWS // NW


def _sc_aggregate(h, src1d, dst1d, attr2d):
    mesh = plsc.VectorSubcoreMesh(core_axis_name="c", subcore_axis_name="s")

    @functools.partial(
        pl.kernel,
        mesh=mesh,
        out_type=jax.ShapeDtypeStruct((2, N_ACC, D), jnp.float32),
        scratch_types=[
            pltpu.VMEM((2, CHUNK), jnp.int32),
            pltpu.VMEM((2, CHUNK), jnp.int32),
            pltpu.VMEM((2, CHUNK, D), jnp.float32),
            pltpu.VMEM((2, CHUNK, D), jnp.float32),
            pltpu.VMEM_SHARED((N_ACC, D), jnp.float32),
            pltpu.SemaphoreType.DMA,
            pltpu.SemaphoreType.DMA,
            pltpu.SemaphoreType.DMA,
        ],
    )
    def k(h_hbm, src_hbm, dst_hbm, attr_hbm, out_hbm,
          src_v, dst_v, hrows_v, attr_v, aggr_sh, sem_g, sem_a, sem_i):
        c = lax.axis_index("c")
        s = lax.axis_index("s")
        w = c * 16 + s

        def zrow(i, carry2):
            for j in range(D // 16):
                attr_v[0, i, pl.ds(j * 16, 16)] = jnp.zeros((16,),
                                                            jnp.float32)
            return carry2

        lax.fori_loop(0, CHUNK, zrow, 0)
        rows_per_tile = N_ACC // 16
        base = s * rows_per_tile
        for kk in range(rows_per_tile // CHUNK):
            pltpu.sync_copy(attr_v.at[0],
                            aggr_sh.at[pl.ds(base + kk * CHUNK, CHUNK)])
        rem = rows_per_tile % CHUNK
        if rem:
            pltpu.sync_copy(
                attr_v.at[0].at[pl.ds(0, rem)],
                aggr_sh.at[pl.ds(base + rows_per_tile - rem, rem)])
        plsc.subcore_barrier()

        start = w * KPW
        end = start + KPW

        def issue_idx(r):
            p = r % 2
            sl = pl.ds(r * CHUNK, CHUNK)
            pltpu.async_copy(src_hbm.at[sl], src_v.at[p], sem_i)
            pltpu.async_copy(dst_hbm.at[sl], dst_v.at[p], sem_i)

        def wait_idx(r):
            p = r % 2
            sl = pl.ds(r * CHUNK, CHUNK)
            pltpu.make_async_copy(src_hbm.at[sl], src_v.at[p], sem_i).wait()
            pltpu.make_async_copy(dst_hbm.at[sl], dst_v.at[p], sem_i).wait()

        def issue_gather(r):
            p = r % 2
            pltpu.async_copy(h_hbm.at[src_v.at[p]], hrows_v.at[p], sem_g)

        def wait_gather(r):
            p = r % 2
            pltpu.make_async_copy(h_hbm.at[src_v.at[p]], hrows_v.at[p],
                                  sem_g).wait()

        def issue_attr(r):
            p = r % 2
            sl = pl.ds(r * CHUNK, CHUNK)
            pltpu.async_copy(attr_hbm.at[sl], attr_v.at[p], sem_a)

        def wait_attr(r):
            p = r % 2
            sl = pl.ds(r * CHUNK, CHUNK)
            pltpu.make_async_copy(attr_hbm.at[sl], attr_v.at[p],
                                  sem_a).wait()

        p0 = start % 2
        sl0 = pl.ds(start * CHUNK, CHUNK)
        pltpu.sync_copy(src_hbm.at[sl0], src_v.at[p0])
        pltpu.sync_copy(dst_hbm.at[sl0], dst_v.at[p0])
        issue_attr(start)
        issue_gather(start)

        def body(r, carry):
            p = r % 2

            @pl.when(r + 1 < end)
            def _():
                issue_idx(r + 1)
                issue_attr(r + 1)

            wait_attr(r)
            pltpu.sync_copy(attr_v.at[p], aggr_sh.at[dst_v.at[p]], add=True)
            wait_gather(r)

            @pl.when(r + 1 < end)
            def _():
                wait_idx(r + 1)
                issue_gather(r + 1)

            pltpu.sync_copy(hrows_v.at[p], aggr_sh.at[dst_v.at[p]], add=True)
            return carry

        lax.fori_loop(start, end, body, 0)
        plsc.subcore_barrier()

        pltpu.sync_copy(aggr_sh.at[pl.ds(s * rows_per_tile, rows_per_tile)],
                        out_hbm.at[c, pl.ds(s * rows_per_tile, rows_per_tile)])

    return k(h, src1d, dst1d, attr2d)


def _tc_mlp_body(h_ref, p_ref, w1_ref, b1_ref, w2_ref, b2_ref, eps_ref,
                 out_ref):
    aggr = p_ref[0] + p_ref[1]
    hid = jnp.dot(aggr, w1_ref[...], preferred_element_type=jnp.float32)
    hid = jnp.maximum(hid + b1_ref[...], 0.0)
    out = jnp.dot(hid, w2_ref[...], preferred_element_type=jnp.float32)
    out_ref[...] = (1.0 + eps_ref[0]) * h_ref[...] + out + b2_ref[...]


def _tc_mlp(h, partials, W1, b1, W2, b2, eps):
    BR = 256
    grid = (pl.cdiv(N, BR),)
    return pl.pallas_call(
        _tc_mlp_body,
        grid=grid,
        in_specs=[
            pl.BlockSpec((BR, D), lambda i: (i, 0)),
            pl.BlockSpec((2, BR, D), lambda i: (0, i, 0)),
            pl.BlockSpec((D, 2 * D), lambda i: (0, 0)),
            pl.BlockSpec((1, 2 * D), lambda i: (0, 0)),
            pl.BlockSpec((2 * D, D), lambda i: (0, 0)),
            pl.BlockSpec((1, D), lambda i: (0, 0)),
            pl.BlockSpec(memory_space=pltpu.SMEM),
        ],
        out_specs=pl.BlockSpec((BR, D), lambda i: (i, 0)),
        out_shape=jax.ShapeDtypeStruct((N, D), jnp.float32),
    )(h, partials, W1, b1.reshape(1, 2 * D), W2, b2.reshape(1, D), eps)


def kernel(h, edge_index, edge_attr, W1, b1, W2, b2, eps):
    src1d = edge_index[0]
    dst1d = edge_index[1]
    partials = _sc_aggregate(h, src1d, dst1d, edge_attr)
    return _tc_mlp(h, partials, W1, b1, W2, b2, eps)


# speedup vs baseline: 2.7437x; 1.0018x over previous
import functools

import jax
import jax.numpy as jnp
from jax import lax
from jax.experimental import pallas as pl
from jax.experimental.pallas import tpu as pltpu
from jax.experimental.pallas import tpu_sc as plsc

N = 10000
N_ACC = 10112
E = 320000
D = 128
CHUNK = 80
ROWS = E // CHUNK
NW = 32
KPW = ROWS // NW


def _sc_aggregate(h, src1d, dst1d, attr2d):
    mesh = plsc.VectorSubcoreMesh(core_axis_name="c", subcore_axis_name="s")

    @functools.partial(
        pl.kernel,
        mesh=mesh,
        out_type=jax.ShapeDtypeStruct((2, N_ACC, D), jnp.float32),
        scratch_types=[
            pltpu.VMEM((2, CHUNK), jnp.int32),
            pltpu.VMEM((2, CHUNK), jnp.int32),
            pltpu.VMEM((2, CHUNK, D), jnp.float32),
            pltpu.VMEM((2, CHUNK, D), jnp.float32),
            pltpu.VMEM_SHARED((N_ACC, D), jnp.float32),
            pltpu.SemaphoreType.DMA,
            pltpu.SemaphoreType.DMA,
            pltpu.SemaphoreType.DMA,
            pltpu.SemaphoreType.DMA,
        ],
    )
    def k(h_hbm, src_hbm, dst_hbm, attr_hbm, out_hbm,
          src_v, dst_v, hrows_v, attr_v, aggr_sh, sem_g, sem_a, sem_i,
          sem_s):
        c = lax.axis_index("c")
        s = lax.axis_index("s")
        w = c * 16 + s

        def zrow(i, carry2):
            for j in range(D // 16):
                attr_v[0, i, pl.ds(j * 16, 16)] = jnp.zeros((16,),
                                                            jnp.float32)
            return carry2

        lax.fori_loop(0, CHUNK, zrow, 0)
        rows_per_tile = N_ACC // 16
        base = s * rows_per_tile
        for kk in range(rows_per_tile // CHUNK):
            pltpu.sync_copy(attr_v.at[0],
                            aggr_sh.at[pl.ds(base + kk * CHUNK, CHUNK)])
        rem = rows_per_tile % CHUNK
        if rem:
            pltpu.sync_copy(
                attr_v.at[0].at[pl.ds(0, rem)],
                aggr_sh.at[pl.ds(base + rows_per_tile - rem, rem)])
        plsc.subcore_barrier()

        start = w * KPW
        end = start + KPW

        def issue_idx(r):
            p = r % 2
            sl = pl.ds(r * CHUNK, CHUNK)
            pltpu.async_copy(src_hbm.at[sl], src_v.at[p], sem_i)
            pltpu.async_copy(dst_hbm.at[sl], dst_v.at[p], sem_i)

        def wait_idx(r):
            p = r % 2
            sl = pl.ds(r * CHUNK, CHUNK)
            pltpu.make_async_copy(src_hbm.at[sl], src_v.at[p], sem_i).wait()
            pltpu.make_async_copy(dst_hbm.at[sl], dst_v.at[p], sem_i).wait()

        def issue_gather(r):
            p = r % 2
            pltpu.async_copy(h_hbm.at[src_v.at[p]], hrows_v.at[p], sem_g)

        def wait_gather(r):
            p = r % 2
            pltpu.make_async_copy(h_hbm.at[src_v.at[p]], hrows_v.at[p],
                                  sem_g).wait()

        def issue_attr(r):
            p = r % 2
            sl = pl.ds(r * CHUNK, CHUNK)
            pltpu.async_copy(attr_hbm.at[sl], attr_v.at[p], sem_a)

        def wait_attr(r):
            p = r % 2
            sl = pl.ds(r * CHUNK, CHUNK)
            pltpu.make_async_copy(attr_hbm.at[sl], attr_v.at[p],
                                  sem_a).wait()

        p0 = start % 2
        sl0 = pl.ds(start * CHUNK, CHUNK)
        pltpu.sync_copy(src_hbm.at[sl0], src_v.at[p0])
        pltpu.sync_copy(dst_hbm.at[sl0], dst_v.at[p0])
        issue_attr(start)
        issue_gather(start)

        def wait_scatters(r):
            p = r % 2
            pltpu.make_async_copy(attr_v.at[p], aggr_sh.at[dst_v.at[p]],
                                  sem_s).wait()
            pltpu.make_async_copy(hrows_v.at[p], aggr_sh.at[dst_v.at[p]],
                                  sem_s).wait()

        def body(r, carry):
            p = r % 2

            @pl.when(r > start)
            def _():
                wait_scatters(r - 1)

            @pl.when(r + 1 < end)
            def _():
                issue_idx(r + 1)
                issue_attr(r + 1)

            wait_attr(r)
            pltpu.async_copy(attr_v.at[p], aggr_sh.at[dst_v.at[p]], sem_s,
                             add=True)
            wait_gather(r)

            @pl.when(r + 1 < end)
            def _():
                wait_idx(r + 1)
                issue_gather(r + 1)

            pltpu.async_copy(hrows_v.at[p], aggr_sh.at[dst_v.at[p]], sem_s,
                             add=True)
            return carry

        lax.fori_loop(start, end, body, 0)
        wait_scatters(end - 1)
        plsc.subcore_barrier()

        pltpu.sync_copy(aggr_sh.at[pl.ds(s * rows_per_tile, rows_per_tile)],
                        out_hbm.at[c, pl.ds(s * rows_per_tile, rows_per_tile)])

    return k(h, src1d, dst1d, attr2d)


def _tc_mlp_body(h_ref, p_ref, w1_ref, b1_ref, w2_ref, b2_ref, eps_ref,
                 out_ref):
    aggr = p_ref[0] + p_ref[1]
    hid = jnp.dot(aggr, w1_ref[...], preferred_element_type=jnp.float32)
    hid = jnp.maximum(hid + b1_ref[...], 0.0)
    out = jnp.dot(hid, w2_ref[...], preferred_element_type=jnp.float32)
    out_ref[...] = (1.0 + eps_ref[0]) * h_ref[...] + out + b2_ref[...]


def _tc_mlp(h, partials, W1, b1, W2, b2, eps):
    BR = 256
    grid = (pl.cdiv(N, BR),)
    return pl.pallas_call(
        _tc_mlp_body,
        grid=grid,
        in_specs=[
            pl.BlockSpec((BR, D), lambda i: (i, 0)),
            pl.BlockSpec((2, BR, D), lambda i: (0, i, 0)),
            pl.BlockSpec((D, 2 * D), lambda i: (0, 0)),
            pl.BlockSpec((1, 2 * D), lambda i: (0, 0)),
            pl.BlockSpec((2 * D, D), lambda i: (0, 0)),
            pl.BlockSpec((1, D), lambda i: (0, 0)),
            pl.BlockSpec(memory_space=pltpu.SMEM),
        ],
        out_specs=pl.BlockSpec((BR, D), lambda i: (i, 0)),
        out_shape=jax.ShapeDtypeStruct((N, D), jnp.float32),
    )(h, partials, W1, b1.reshape(1, 2 * D), W2, b2.reshape(1, D), eps)


def kernel(h, edge_index, edge_attr, W1, b1, W2, b2, eps):
    src1d = edge_index[0]
    dst1d = edge_index[1]
    partials = _sc_aggregate(h, src1d, dst1d, edge_attr)
    return _tc_mlp(h, partials, W1, b1, W2, b2, eps)


# speedup vs baseline: 3.0083x; 1.0965x over previous
import functools

import jax
import jax.numpy as jnp
from jax import lax
from jax.experimental import pallas as pl
from jax.experimental.pallas import tpu as pltpu
from jax.experimental.pallas import tpu_sc as plsc

N = 10000
N_ACC = 10112
E = 320000
D = 128
CHUNK = 80
ROWS = E // CHUNK
NW = 32
KPW = ROWS // NW


def _sc_aggregate(h, ei1d, attr2d):
    mesh = plsc.VectorSubcoreMesh(core_axis_name="c", subcore_axis_name="s")

    @functools.partial(
        pl.kernel,
        mesh=mesh,
        out_type=jax.ShapeDtypeStruct((2, N_ACC, D), jnp.float32),
        scratch_types=[
            pltpu.VMEM((2, CHUNK), jnp.int32),
            pltpu.VMEM((2, CHUNK), jnp.int32),
            pltpu.VMEM((2, CHUNK, D), jnp.float32),
            pltpu.VMEM((2, CHUNK, D), jnp.float32),
            pltpu.VMEM_SHARED((N_ACC, D), jnp.float32),
            pltpu.SemaphoreType.DMA,
            pltpu.SemaphoreType.DMA,
            pltpu.SemaphoreType.DMA,
            pltpu.SemaphoreType.DMA,
        ],
    )
    def k(h_hbm, ei_hbm, attr_hbm, out_hbm,
          src_v, dst_v, hrows_v, attr_v, aggr_sh, sem_g, sem_a, sem_i,
          sem_s):
        c = lax.axis_index("c")
        s = lax.axis_index("s")
        w = c * 16 + s

        def zrow(i, carry2):
            for j in range(D // 16):
                attr_v[0, i, pl.ds(j * 16, 16)] = jnp.zeros((16,),
                                                            jnp.float32)
            return carry2

        lax.fori_loop(0, CHUNK, zrow, 0)
        rows_per_tile = N_ACC // 16
        base = s * rows_per_tile
        for kk in range(rows_per_tile // CHUNK):
            pltpu.sync_copy(attr_v.at[0],
                            aggr_sh.at[pl.ds(base + kk * CHUNK, CHUNK)])
        rem = rows_per_tile % CHUNK
        if rem:
            pltpu.sync_copy(
                attr_v.at[0].at[pl.ds(0, rem)],
                aggr_sh.at[pl.ds(base + rows_per_tile - rem, rem)])
        plsc.subcore_barrier()

        start = w * KPW
        end = start + KPW

        def issue_idx(r):
            p = r % 2
            ssl = pl.ds(r * CHUNK, CHUNK)
            dsl = pl.ds(E + r * CHUNK, CHUNK)
            pltpu.async_copy(ei_hbm.at[ssl], src_v.at[p], sem_i)
            pltpu.async_copy(ei_hbm.at[dsl], dst_v.at[p], sem_i)

        def wait_idx(r):
            p = r % 2
            ssl = pl.ds(r * CHUNK, CHUNK)
            dsl = pl.ds(E + r * CHUNK, CHUNK)
            pltpu.make_async_copy(ei_hbm.at[ssl], src_v.at[p], sem_i).wait()
            pltpu.make_async_copy(ei_hbm.at[dsl], dst_v.at[p], sem_i).wait()

        def issue_gather(r):
            p = r % 2
            pltpu.async_copy(h_hbm.at[src_v.at[p]], hrows_v.at[p], sem_g)

        def wait_gather(r):
            p = r % 2
            pltpu.make_async_copy(h_hbm.at[src_v.at[p]], hrows_v.at[p],
                                  sem_g).wait()

        def issue_attr(r):
            p = r % 2
            sl = pl.ds(r * CHUNK, CHUNK)
            pltpu.async_copy(attr_hbm.at[sl], attr_v.at[p], sem_a)

        def wait_attr(r):
            p = r % 2
            sl = pl.ds(r * CHUNK, CHUNK)
            pltpu.make_async_copy(attr_hbm.at[sl], attr_v.at[p],
                                  sem_a).wait()

        p0 = start % 2
        pltpu.sync_copy(ei_hbm.at[pl.ds(start * CHUNK, CHUNK)],
                        src_v.at[p0])
        pltpu.sync_copy(ei_hbm.at[pl.ds(E + start * CHUNK, CHUNK)],
                        dst_v.at[p0])
        issue_attr(start)
        issue_gather(start)

        def wait_scatters(r):
            p = r % 2
            pltpu.make_async_copy(attr_v.at[p], aggr_sh.at[dst_v.at[p]],
                                  sem_s).wait()
            pltpu.make_async_copy(hrows_v.at[p], aggr_sh.at[dst_v.at[p]],
                                  sem_s).wait()

        def body(r, carry):
            p = r % 2

            @pl.when(r > start)
            def _():
                wait_scatters(r - 1)

            @pl.when(r + 1 < end)
            def _():
                issue_idx(r + 1)
                issue_attr(r + 1)

            wait_attr(r)
            pltpu.async_copy(attr_v.at[p], aggr_sh.at[dst_v.at[p]], sem_s,
                             add=True)
            wait_gather(r)

            @pl.when(r + 1 < end)
            def _():
                wait_idx(r + 1)
                issue_gather(r + 1)

            pltpu.async_copy(hrows_v.at[p], aggr_sh.at[dst_v.at[p]], sem_s,
                             add=True)
            return carry

        lax.fori_loop(start, end, body, 0)
        wait_scatters(end - 1)
        plsc.subcore_barrier()

        pltpu.sync_copy(aggr_sh.at[pl.ds(s * rows_per_tile, rows_per_tile)],
                        out_hbm.at[c, pl.ds(s * rows_per_tile, rows_per_tile)])

    return k(h, ei1d, attr2d)


def _tc_mlp_body(h_ref, p_ref, w1_ref, b1_ref, w2_ref, b2_ref, eps_ref,
                 out_ref):
    aggr = p_ref[0] + p_ref[1]
    hid = jnp.dot(aggr, w1_ref[...], preferred_element_type=jnp.float32)
    hid = jnp.maximum(hid + b1_ref[...], 0.0)
    out = jnp.dot(hid, w2_ref[...], preferred_element_type=jnp.float32)
    out_ref[...] = (1.0 + eps_ref[0]) * h_ref[...] + out + b2_ref[...]


def _tc_mlp(h, partials, W1, b1, W2, b2, eps):
    BR = 512
    grid = (pl.cdiv(N, BR),)
    return pl.pallas_call(
        _tc_mlp_body,
        grid=grid,
        in_specs=[
            pl.BlockSpec((BR, D), lambda i: (i, 0)),
            pl.BlockSpec((2, BR, D), lambda i: (0, i, 0)),
            pl.BlockSpec((D, 2 * D), lambda i: (0, 0)),
            pl.BlockSpec((1, 2 * D), lambda i: (0, 0)),
            pl.BlockSpec((2 * D, D), lambda i: (0, 0)),
            pl.BlockSpec((1, D), lambda i: (0, 0)),
            pl.BlockSpec(memory_space=pltpu.SMEM),
        ],
        out_specs=pl.BlockSpec((BR, D), lambda i: (i, 0)),
        out_shape=jax.ShapeDtypeStruct((N, D), jnp.float32),
    )(h, partials, W1, b1.reshape(1, 2 * D), W2, b2.reshape(1, D), eps)


def kernel(h, edge_index, edge_attr, W1, b1, W2, b2, eps):
    ei1d = edge_index.reshape(2 * E)
    partials = _sc_aggregate(h, ei1d, edge_attr)
    return _tc_mlp(h, partials, W1, b1, W2, b2, eps)
